# baseline (speedup 1.0000x reference)
import functools

import jax
import jax.numpy as jnp
from jax import lax
from jax.experimental import pallas as pl
from jax.experimental.pallas import tpu as pltpu
from jax.experimental.pallas import tpu_sc as plsc

N = 10000
D = 128
E = 320000

NC = 2
NS = 16
NW = NC * NS
CHUNK = 64
EPT = E // NW
NCHUNK = -(-EPT // CHUNK)
EPT_PAD = NCHUNK * CHUNK
E_PAD = EPT_PAD * NW
NPAD = 10240
RPS = NPAD // NS


def _sc_deg_body(dst_hbm, zeros_hbm, out_hbm, idx_d, ones_v, acc, sem):
    c = lax.axis_index("c")
    s = lax.axis_index("s")
    wid = s * NC + c
    pltpu.sync_copy(zeros_hbm.at[pl.ds(s * RPS, RPS)], acc.at[pl.ds(s * RPS, RPS)])
    pltpu.sync_copy(dst_hbm.at[wid], idx_d)
    for i in range(CHUNK // 16):
        ones_v[pl.ds(i * 16, 16)] = jnp.full((16,), 1.0, jnp.float32)
    plsc.subcore_barrier()
    DLAG = 16
    dd = []
    for j in range(NCHUNK):
        dd.append(pltpu.async_copy(ones_v, acc.at[idx_d.at[j, 0]], sem,
                                   add=True))
        if j >= DLAG:
            dd[j - DLAG].wait()
    for j in range(max(0, NCHUNK - DLAG), NCHUNK):
        dd[j].wait()
    plsc.subcore_barrier()
    pltpu.sync_copy(acc.at[pl.ds(s * RPS, RPS)], out_hbm.at[c, pl.ds(s * RPS, RPS)])


_sc_deg = pl.kernel(
    _sc_deg_body,
    out_type=jax.ShapeDtypeStruct((NC, NPAD), jnp.float32),
    mesh=plsc.VectorSubcoreMesh(core_axis_name="c", subcore_axis_name="s"),
    scratch_types=[
        pltpu.VMEM((NCHUNK, 1, CHUNK), jnp.int32),
        pltpu.VMEM((CHUNK,), jnp.float32),
        pltpu.VMEM_SHARED((NPAD,), jnp.float32),
        pltpu.SemaphoreType.DMA,
    ],
)


IBUF = 8
RBUF = 5
GLEAD = RBUF - 1

NCH0 = 83
NCH1 = 230
PCH = NCH0 + NCH1
NCHMAX = max(NCH0, NCH1)
EPAIR_PAD = PCH * CHUNK


def _sc_scatter_body(src_hbm, dst_hbm, table_hbm, zeros_hbm, out_hbm,
                     idx_s, idx_d, rows, acc, isems_s, isems_d, gsems, ssems):
    c = lax.axis_index("c")
    s = lax.axis_index("s")
    pltpu.sync_copy(zeros_hbm.at[pl.ds(s * RPS, RPS)], acc.at[pl.ds(s * RPS, RPS)])
    plsc.subcore_barrier()

    def pipeline(nch, base):
        id_s = [None] * IBUF
        id_d = [None] * IBUF
        gd = [None] * RBUF
        sd = [None] * RBUF

        def issue_idx(j):
            b = j % IBUF
            id_s[b] = pltpu.async_copy(src_hbm.at[s, base + j], idx_s.at[b],
                                       isems_s[b])
            id_d[b] = pltpu.async_copy(dst_hbm.at[s, base + j], idx_d.at[b],
                                       isems_d[b])

        def issue_gather(j):
            b = j % IBUF
            r = j % RBUF
            if sd[r] is not None:
                sd[r].wait()
                sd[r] = None
            id_s[b].wait()
            gd[r] = pltpu.async_copy(table_hbm.at[idx_s.at[b, 0]], rows.at[r],
                                     gsems[r])

        for j in range(min(IBUF - 1, nch)):
            issue_idx(j)
        for j in range(min(GLEAD, nch)):
            issue_gather(j)
        for j in range(nch):
            r = j % RBUF
            if j + IBUF - 1 < nch:
                issue_idx(j + IBUF - 1)
            if j + GLEAD < nch:
                issue_gather(j + GLEAD)
            gd[r].wait()
            id_d[j % IBUF].wait()
            sd[r] = pltpu.async_copy(rows.at[r], acc.at[idx_d.at[j % IBUF, 0]],
                                     ssems[r], add=True)
        for r in range(RBUF):
            if sd[r] is not None:
                sd[r].wait()

    @pl.when(c == 0)
    def _():
        pipeline(NCH0, 0)

    @pl.when(c == 1)
    def _():
        pipeline(NCH1, NCH0)

    plsc.subcore_barrier()
    pltpu.sync_copy(acc.at[pl.ds(s * RPS, RPS)], out_hbm.at[c, pl.ds(s * RPS, RPS)])


_sc_scatter = pl.kernel(
    _sc_scatter_body,
    out_type=jax.ShapeDtypeStruct((NC, NPAD, D), jnp.float32),
    mesh=plsc.VectorSubcoreMesh(core_axis_name="c", subcore_axis_name="s"),
    scratch_types=[
        pltpu.VMEM((IBUF, 1, CHUNK), jnp.int32),
        pltpu.VMEM((IBUF, 1, CHUNK), jnp.int32),
        pltpu.VMEM((RBUF, CHUNK, D), jnp.float32),
        pltpu.VMEM_SHARED((NPAD, D), jnp.float32),
        [pltpu.SemaphoreType.DMA] * IBUF,
        [pltpu.SemaphoreType.DMA] * IBUF,
        [pltpu.SemaphoreType.DMA] * RBUF,
        [pltpu.SemaphoreType.DMA] * RBUF,
    ],
)


def _tc_pre_body(x_ref, w_ref, b_ref, degp_ref, hs_ref, isd_ref):
    deg = degp_ref[0] + degp_ref[1]
    isd = lax.rsqrt(deg + 1.0)
    isd_ref[...] = isd
    m = jnp.dot(x_ref[...], w_ref[...], preferred_element_type=jnp.float32)
    hs_ref[...] = (m + b_ref[...]) * isd[:N]


_tc_pre = pl.pallas_call(
    _tc_pre_body,
    out_shape=[
        jax.ShapeDtypeStruct((N, D), jnp.float32),
        jax.ShapeDtypeStruct((NPAD, 1), jnp.float32),
    ],
)


def _bn_relu(agg, g, be):
    mean = jnp.mean(agg, axis=0)
    var = jnp.mean(agg * agg, axis=0) - mean * mean
    return jnp.maximum((agg - mean) * lax.rsqrt(var + 1e-5) * g + be, 0.0)


def _tc_mid_body(p_ref, hs_ref, isd_ref, g_ref, be_ref, w_ref, b_ref,
                 h1_ref, hs2_ref):
    isd = isd_ref[:N]
    agg = (p_ref[0, :N] + p_ref[1, :N] + hs_ref[...]) * isd
    h1 = _bn_relu(agg, g_ref[...], be_ref[...])
    h1_ref[...] = h1
    m2 = jnp.dot(h1, w_ref[...], preferred_element_type=jnp.float32)
    hs2_ref[...] = (m2 + b_ref[...]) * isd


_tc_mid = pl.pallas_call(
    _tc_mid_body,
    out_shape=[
        jax.ShapeDtypeStruct((N, D), jnp.float32),
        jax.ShapeDtypeStruct((N, D), jnp.float32),
    ],
)


def _tc_fin_body(p_ref, hs_ref, isd_ref, g_ref, be_ref, h1_ref, out_ref):
    agg = (p_ref[0, :N] + p_ref[1, :N] + hs_ref[...]) * isd_ref[:N]
    out_ref[...] = _bn_relu(agg, g_ref[...], be_ref[...]) + h1_ref[...]


_tc_fin = pl.pallas_call(
    _tc_fin_body,
    out_shape=jax.ShapeDtypeStruct((N, D), jnp.float32),
)


def kernel(x, edge_index, W1, b1, g1, be1, W2, b2, g2, be2):
    src = edge_index[0]
    dst = edge_index[1]
    dst_pd = jnp.pad(dst, (0, E_PAD - E), constant_values=N).reshape(NW, NCHUNK, 1, CHUNK)
    src2 = src.reshape(NS, 2 * EPT)
    dst2 = dst.reshape(NS, 2 * EPT)
    pad2 = ((0, 0), (0, EPAIR_PAD - 2 * EPT))
    src_pp = jnp.pad(src2, pad2).reshape(NS, PCH, 1, CHUNK)
    dst_pp = jnp.pad(dst2, pad2, constant_values=N).reshape(NS, PCH, 1, CHUNK)
    z1 = jnp.zeros((NPAD,), jnp.float32)
    z2 = jnp.zeros((NPAD, D), jnp.float32)

    degp = _sc_deg(dst_pd, z1)
    hs1, isd = _tc_pre(x, W1, b1, degp.reshape(NC, NPAD, 1))
    p1 = _sc_scatter(src_pp, dst_pp, hs1, z2)
    h1, hs2 = _tc_mid(p1, hs1, isd, g1, be1, W2, b2)
    p2 = _sc_scatter(src_pp, dst_pp, hs2, z2)
    return _tc_fin(p2, hs2, isd, g2, be2, h1)


# speedup vs baseline: 1.0427x; 1.0427x over previous
import functools

import jax
import jax.numpy as jnp
from jax import lax
from jax.experimental import pallas as pl
from jax.experimental.pallas import tpu as pltpu
from jax.experimental.pallas import tpu_sc as plsc

N = 10000
D = 128
E = 320000

NC = 2
NS = 16
NW = NC * NS
CHUNK = 64
EPT = E // NW
NCHUNK = -(-EPT // CHUNK)
EPT_PAD = NCHUNK * CHUNK
E_PAD = EPT_PAD * NW
NPAD = 10240
RPS = NPAD // NS


def _sc_deg_body(dst_hbm, zeros_hbm, out_hbm, idx_d, ones_v, acc, sem):
    c = lax.axis_index("c")
    s = lax.axis_index("s")
    wid = s * NC + c
    pltpu.sync_copy(zeros_hbm.at[pl.ds(s * RPS, RPS)], acc.at[pl.ds(s * RPS, RPS)])
    pltpu.sync_copy(dst_hbm.at[wid], idx_d)
    for i in range(CHUNK // 16):
        ones_v[pl.ds(i * 16, 16)] = jnp.full((16,), 1.0, jnp.float32)
    plsc.subcore_barrier()
    DLAG = 16
    dd = []
    for j in range(NCHUNK):
        dd.append(pltpu.async_copy(ones_v, acc.at[idx_d.at[j, 0]], sem,
                                   add=True))
        if j >= DLAG:
            dd[j - DLAG].wait()
    for j in range(max(0, NCHUNK - DLAG), NCHUNK):
        dd[j].wait()
    plsc.subcore_barrier()
    pltpu.sync_copy(acc.at[pl.ds(s * RPS, RPS)], out_hbm.at[c, pl.ds(s * RPS, RPS)])


_sc_deg = pl.kernel(
    _sc_deg_body,
    out_type=jax.ShapeDtypeStruct((NC, NPAD), jnp.float32),
    mesh=plsc.VectorSubcoreMesh(core_axis_name="c", subcore_axis_name="s"),
    scratch_types=[
        pltpu.VMEM((NCHUNK, 1, CHUNK), jnp.int32),
        pltpu.VMEM((CHUNK,), jnp.float32),
        pltpu.VMEM_SHARED((NPAD,), jnp.float32),
        pltpu.SemaphoreType.DMA,
    ],
)


IBUF = 8
RBUF = 5
GLEAD = RBUF - 1

NCH0 = 98
NCH1 = 215
PCH = NCH0 + NCH1
NCHMAX = max(NCH0, NCH1)
EPAIR_PAD = PCH * CHUNK


def _sc_scatter_body(src_hbm, dst_hbm, table_hbm, zeros_hbm, out_hbm,
                     idx_s, idx_d, rows, acc, isems_s, isems_d, gsems, ssems):
    c = lax.axis_index("c")
    s = lax.axis_index("s")
    pltpu.sync_copy(zeros_hbm.at[pl.ds(s * RPS, RPS)], acc.at[pl.ds(s * RPS, RPS)])
    plsc.subcore_barrier()

    def pipeline(nch, base):
        id_s = [None] * IBUF
        id_d = [None] * IBUF
        gd = [None] * RBUF
        sd = [None] * RBUF

        def issue_idx(j):
            b = j % IBUF
            id_s[b] = pltpu.async_copy(src_hbm.at[s, base + j], idx_s.at[b],
                                       isems_s[b])
            id_d[b] = pltpu.async_copy(dst_hbm.at[s, base + j], idx_d.at[b],
                                       isems_d[b])

        def issue_gather(j):
            b = j % IBUF
            r = j % RBUF
            if sd[r] is not None:
                sd[r].wait()
                sd[r] = None
            id_s[b].wait()
            gd[r] = pltpu.async_copy(table_hbm.at[idx_s.at[b, 0]], rows.at[r],
                                     gsems[r])

        for j in range(min(IBUF - 1, nch)):
            issue_idx(j)
        for j in range(min(GLEAD, nch)):
            issue_gather(j)
        for j in range(nch):
            r = j % RBUF
            if j + IBUF - 1 < nch:
                issue_idx(j + IBUF - 1)
            if j + GLEAD < nch:
                issue_gather(j + GLEAD)
            gd[r].wait()
            id_d[j % IBUF].wait()
            sd[r] = pltpu.async_copy(rows.at[r], acc.at[idx_d.at[j % IBUF, 0]],
                                     ssems[r], add=True)
        for r in range(RBUF):
            if sd[r] is not None:
                sd[r].wait()

    @pl.when(c == 0)
    def _():
        pipeline(NCH0, 0)

    @pl.when(c == 1)
    def _():
        pipeline(NCH1, NCH0)

    plsc.subcore_barrier()
    pltpu.sync_copy(acc.at[pl.ds(s * RPS, RPS)], out_hbm.at[c, pl.ds(s * RPS, RPS)])


_sc_scatter = pl.kernel(
    _sc_scatter_body,
    out_type=jax.ShapeDtypeStruct((NC, NPAD, D), jnp.float32),
    mesh=plsc.VectorSubcoreMesh(core_axis_name="c", subcore_axis_name="s"),
    scratch_types=[
        pltpu.VMEM((IBUF, 1, CHUNK), jnp.int32),
        pltpu.VMEM((IBUF, 1, CHUNK), jnp.int32),
        pltpu.VMEM((RBUF, CHUNK, D), jnp.float32),
        pltpu.VMEM_SHARED((NPAD, D), jnp.float32),
        [pltpu.SemaphoreType.DMA] * IBUF,
        [pltpu.SemaphoreType.DMA] * IBUF,
        [pltpu.SemaphoreType.DMA] * RBUF,
        [pltpu.SemaphoreType.DMA] * RBUF,
    ],
)


def _tc_pre_body(x_ref, w_ref, b_ref, degp_ref, hs_ref, isd_ref):
    deg = degp_ref[0] + degp_ref[1]
    isd = lax.rsqrt(deg + 1.0)
    isd_ref[...] = isd
    m = jnp.dot(x_ref[...], w_ref[...], preferred_element_type=jnp.float32)
    hs_ref[...] = (m + b_ref[...]) * isd[:N]


_tc_pre = pl.pallas_call(
    _tc_pre_body,
    out_shape=[
        jax.ShapeDtypeStruct((N, D), jnp.float32),
        jax.ShapeDtypeStruct((NPAD, 1), jnp.float32),
    ],
)


def _bn_relu(agg, g, be):
    mean = jnp.mean(agg, axis=0)
    var = jnp.mean(agg * agg, axis=0) - mean * mean
    return jnp.maximum((agg - mean) * lax.rsqrt(var + 1e-5) * g + be, 0.0)


def _tc_mid_body(p_ref, hs_ref, isd_ref, g_ref, be_ref, w_ref, b_ref,
                 h1_ref, hs2_ref):
    isd = isd_ref[:N]
    agg = (p_ref[0, :N] + p_ref[1, :N] + hs_ref[...]) * isd
    h1 = _bn_relu(agg, g_ref[...], be_ref[...])
    h1_ref[...] = h1
    m2 = jnp.dot(h1, w_ref[...], preferred_element_type=jnp.float32)
    hs2_ref[...] = (m2 + b_ref[...]) * isd


_tc_mid = pl.pallas_call(
    _tc_mid_body,
    out_shape=[
        jax.ShapeDtypeStruct((N, D), jnp.float32),
        jax.ShapeDtypeStruct((N, D), jnp.float32),
    ],
)


def _tc_fin_body(p_ref, hs_ref, isd_ref, g_ref, be_ref, h1_ref, out_ref):
    agg = (p_ref[0, :N] + p_ref[1, :N] + hs_ref[...]) * isd_ref[:N]
    out_ref[...] = _bn_relu(agg, g_ref[...], be_ref[...]) + h1_ref[...]


_tc_fin = pl.pallas_call(
    _tc_fin_body,
    out_shape=jax.ShapeDtypeStruct((N, D), jnp.float32),
)


def kernel(x, edge_index, W1, b1, g1, be1, W2, b2, g2, be2):
    src = edge_index[0]
    dst = edge_index[1]
    dst_pd = jnp.pad(dst, (0, E_PAD - E), constant_values=N).reshape(NW, NCHUNK, 1, CHUNK)
    src2 = src.reshape(NS, 2 * EPT)
    dst2 = dst.reshape(NS, 2 * EPT)
    pad2 = ((0, 0), (0, EPAIR_PAD - 2 * EPT))
    src_pp = jnp.pad(src2, pad2).reshape(NS, PCH, 1, CHUNK)
    dst_pp = jnp.pad(dst2, pad2, constant_values=N).reshape(NS, PCH, 1, CHUNK)
    z1 = jnp.zeros((NPAD,), jnp.float32)
    z2 = jnp.zeros((NPAD, D), jnp.float32)

    degp = _sc_deg(dst_pd, z1)
    hs1, isd = _tc_pre(x, W1, b1, degp.reshape(NC, NPAD, 1))
    p1 = _sc_scatter(src_pp, dst_pp, hs1, z2)
    h1, hs2 = _tc_mid(p1, hs1, isd, g1, be1, W2, b2)
    p2 = _sc_scatter(src_pp, dst_pp, hs2, z2)
    return _tc_fin(p2, hs2, isd, g2, be2, h1)


# speedup vs baseline: 1.1184x; 1.0726x over previous
import functools

import jax
import jax.numpy as jnp
from jax import lax
from jax.experimental import pallas as pl
from jax.experimental.pallas import tpu as pltpu
from jax.experimental.pallas import tpu_sc as plsc

N = 10000
D = 128
E = 320000

NC = 2
NS = 16
NW = NC * NS
CHUNK = 64
EPT = E // NW
NCHUNK = -(-EPT // CHUNK)
EPT_PAD = NCHUNK * CHUNK
E_PAD = EPT_PAD * NW
NPAD = 10240
RPS = NPAD // NS


def _sc_deg_body(dst_hbm, zeros_hbm, out_hbm, idx_d, ones_v, acc, sem):
    c = lax.axis_index("c")
    s = lax.axis_index("s")
    wid = s * NC + c
    pltpu.sync_copy(zeros_hbm.at[pl.ds(s * RPS, RPS)], acc.at[pl.ds(s * RPS, RPS)])
    pltpu.sync_copy(dst_hbm.at[wid], idx_d)
    for i in range(CHUNK // 16):
        ones_v[pl.ds(i * 16, 16)] = jnp.full((16,), 1.0, jnp.float32)
    plsc.subcore_barrier()
    DLAG = 16
    dd = []
    for j in range(NCHUNK):
        dd.append(pltpu.async_copy(ones_v, acc.at[idx_d.at[j, 0]], sem,
                                   add=True))
        if j >= DLAG:
            dd[j - DLAG].wait()
    for j in range(max(0, NCHUNK - DLAG), NCHUNK):
        dd[j].wait()
    plsc.subcore_barrier()
    pltpu.sync_copy(acc.at[pl.ds(s * RPS, RPS)], out_hbm.at[c, pl.ds(s * RPS, RPS)])


_sc_deg = pl.kernel(
    _sc_deg_body,
    out_type=jax.ShapeDtypeStruct((NC, NPAD), jnp.float32),
    mesh=plsc.VectorSubcoreMesh(core_axis_name="c", subcore_axis_name="s"),
    scratch_types=[
        pltpu.VMEM((NCHUNK, 1, CHUNK), jnp.int32),
        pltpu.VMEM((CHUNK,), jnp.float32),
        pltpu.VMEM_SHARED((NPAD,), jnp.float32),
        pltpu.SemaphoreType.DMA,
    ],
)


IBUF = 8
RBUF = 5
GLEAD = RBUF - 1

NCH0 = 125
NCH1 = 188
PCH = NCH0 + NCH1
NCHMAX = max(NCH0, NCH1)
EPAIR_PAD = PCH * CHUNK


def _sc_scatter_body(src_hbm, dst_hbm, table_hbm, zeros_hbm, out_hbm,
                     idx_s, idx_d, rows, acc, isems_s, isems_d, gsems, ssems):
    c = lax.axis_index("c")
    s = lax.axis_index("s")
    pltpu.sync_copy(zeros_hbm.at[pl.ds(s * RPS, RPS)], acc.at[pl.ds(s * RPS, RPS)])
    plsc.subcore_barrier()

    def pipeline(nch, base):
        id_s = [None] * IBUF
        id_d = [None] * IBUF
        gd = [None] * RBUF
        sd = [None] * RBUF

        def issue_idx(j):
            b = j % IBUF
            id_s[b] = pltpu.async_copy(src_hbm.at[s, base + j], idx_s.at[b],
                                       isems_s[b])
            id_d[b] = pltpu.async_copy(dst_hbm.at[s, base + j], idx_d.at[b],
                                       isems_d[b])

        def issue_gather(j):
            b = j % IBUF
            r = j % RBUF
            if sd[r] is not None:
                sd[r].wait()
                sd[r] = None
            id_s[b].wait()
            gd[r] = pltpu.async_copy(table_hbm.at[idx_s.at[b, 0]], rows.at[r],
                                     gsems[r])

        for j in range(min(IBUF - 1, nch)):
            issue_idx(j)
        for j in range(min(GLEAD, nch)):
            issue_gather(j)
        for j in range(nch):
            r = j % RBUF
            if j + IBUF - 1 < nch:
                issue_idx(j + IBUF - 1)
            if j + GLEAD < nch:
                issue_gather(j + GLEAD)
            gd[r].wait()
            id_d[j % IBUF].wait()
            sd[r] = pltpu.async_copy(rows.at[r], acc.at[idx_d.at[j % IBUF, 0]],
                                     ssems[r], add=True)
        for r in range(RBUF):
            if sd[r] is not None:
                sd[r].wait()

    @pl.when(c == 0)
    def _():
        pipeline(NCH0, 0)

    @pl.when(c == 1)
    def _():
        pipeline(NCH1, NCH0)

    plsc.subcore_barrier()
    pltpu.sync_copy(acc.at[pl.ds(s * RPS, RPS)], out_hbm.at[c, pl.ds(s * RPS, RPS)])


_sc_scatter = pl.kernel(
    _sc_scatter_body,
    out_type=jax.ShapeDtypeStruct((NC, NPAD, D), jnp.float32),
    mesh=plsc.VectorSubcoreMesh(core_axis_name="c", subcore_axis_name="s"),
    scratch_types=[
        pltpu.VMEM((IBUF, 1, CHUNK), jnp.int32),
        pltpu.VMEM((IBUF, 1, CHUNK), jnp.int32),
        pltpu.VMEM((RBUF, CHUNK, D), jnp.float32),
        pltpu.VMEM_SHARED((NPAD, D), jnp.float32),
        [pltpu.SemaphoreType.DMA] * IBUF,
        [pltpu.SemaphoreType.DMA] * IBUF,
        [pltpu.SemaphoreType.DMA] * RBUF,
        [pltpu.SemaphoreType.DMA] * RBUF,
    ],
)


def _tc_pre_body(x_ref, w_ref, b_ref, degp_ref, hs_ref, isd_ref):
    deg = degp_ref[0] + degp_ref[1]
    isd = lax.rsqrt(deg + 1.0)
    isd_ref[...] = isd
    m = jnp.dot(x_ref[...], w_ref[...], preferred_element_type=jnp.float32)
    hs_ref[...] = (m + b_ref[...]) * isd[:N]


_tc_pre = pl.pallas_call(
    _tc_pre_body,
    out_shape=[
        jax.ShapeDtypeStruct((N, D), jnp.float32),
        jax.ShapeDtypeStruct((NPAD, 1), jnp.float32),
    ],
)


def _bn_relu(agg, g, be):
    mean = jnp.mean(agg, axis=0)
    var = jnp.mean(agg * agg, axis=0) - mean * mean
    return jnp.maximum((agg - mean) * lax.rsqrt(var + 1e-5) * g + be, 0.0)


def _tc_mid_body(p_ref, hs_ref, isd_ref, g_ref, be_ref, w_ref, b_ref,
                 h1_ref, hs2_ref):
    isd = isd_ref[:N]
    agg = (p_ref[0, :N] + p_ref[1, :N] + hs_ref[...]) * isd
    h1 = _bn_relu(agg, g_ref[...], be_ref[...])
    h1_ref[...] = h1
    m2 = jnp.dot(h1, w_ref[...], preferred_element_type=jnp.float32)
    hs2_ref[...] = (m2 + b_ref[...]) * isd


_tc_mid = pl.pallas_call(
    _tc_mid_body,
    out_shape=[
        jax.ShapeDtypeStruct((N, D), jnp.float32),
        jax.ShapeDtypeStruct((N, D), jnp.float32),
    ],
)


def _tc_fin_body(p_ref, hs_ref, isd_ref, g_ref, be_ref, h1_ref, out_ref):
    agg = (p_ref[0, :N] + p_ref[1, :N] + hs_ref[...]) * isd_ref[:N]
    out_ref[...] = _bn_relu(agg, g_ref[...], be_ref[...]) + h1_ref[...]


_tc_fin = pl.pallas_call(
    _tc_fin_body,
    out_shape=jax.ShapeDtypeStruct((N, D), jnp.float32),
)


def kernel(x, edge_index, W1, b1, g1, be1, W2, b2, g2, be2):
    src = edge_index[0]
    dst = edge_index[1]
    dst_pd = jnp.pad(dst, (0, E_PAD - E), constant_values=N).reshape(NW, NCHUNK, 1, CHUNK)
    src2 = src.reshape(NS, 2 * EPT)
    dst2 = dst.reshape(NS, 2 * EPT)
    pad2 = ((0, 0), (0, EPAIR_PAD - 2 * EPT))
    src_pp = jnp.pad(src2, pad2).reshape(NS, PCH, 1, CHUNK)
    dst_pp = jnp.pad(dst2, pad2, constant_values=N).reshape(NS, PCH, 1, CHUNK)
    z1 = jnp.zeros((NPAD,), jnp.float32)
    z2 = jnp.zeros((NPAD, D), jnp.float32)

    degp = _sc_deg(dst_pd, z1)
    hs1, isd = _tc_pre(x, W1, b1, degp.reshape(NC, NPAD, 1))
    p1 = _sc_scatter(src_pp, dst_pp, hs1, z2)
    h1, hs2 = _tc_mid(p1, hs1, isd, g1, be1, W2, b2)
    p2 = _sc_scatter(src_pp, dst_pp, hs2, z2)
    return _tc_fin(p2, hs2, isd, g2, be2, h1)
